# baseline (speedup 1.0000x reference)
import jax
import jax.numpy as jnp
from jax import lax
from jax.experimental import pallas as pl
from jax.experimental.pallas import tpu as pltpu
from jax.experimental.pallas import tpu_sc as plsc

BATCH = 4096
SEQ = 200
D = 64
NC = 2
NS = 16
NW = NC * NS
ROWS = BATCH * SEQ
RPW = ROWS // NW
BPW = BATCH // NW
CHUNKS = RPW // SEQ
G1 = 104
G2 = SEQ - G1
LANES = 16
NV = (SEQ + LANES - 1) // LANES
XC = SEQ + LANES


def _sc_body(x_h, table2_h, pe_h, out_h, pe_v, xc0, xc1, ip0, ip1,
             pb0, pb1, ob0, ob1, xs0, xs1, g0, g1s, w0, w1):
    xc = (xc0, xc1)
    idxp = (ip0, ip1)
    pbuf = (pb0, pb1)
    obuf = (ob0, ob1)
    xsem = (xs0, xs1)
    gsem = (g0, g1s)
    wsem = (w0, w1)

    wid = lax.axis_index("s") * NC + lax.axis_index("c")
    rbase = wid * RPW
    bbase = wid * BPW

    pltpu.sync_copy(pe_h, pe_v)

    def issue_xc(c, b):
        pltpu.async_copy(x_h.at[pl.ds(rbase + c * SEQ, SEQ)],
                         xc[b].at[pl.ds(0, SEQ)], xsem[b])

    def wait_xc(b):
        pltpu.make_async_copy(x_h.at[pl.ds(rbase, SEQ)],
                              xc[b].at[pl.ds(0, SEQ)], xsem[b]).wait()

    def stage_idx(b):
        for i in range(NV):
            src = pl.ds(i * LANES, LANES)
            idxp[b][src] = lax.shift_right_logical(xc[b][src], 1)

    def issue_gather(b):
        pltpu.async_copy(table2_h.at[idxp[b].at[pl.ds(0, G1)]],
                         pbuf[b].at[pl.ds(0, G1)], gsem[b])
        pltpu.async_copy(table2_h.at[idxp[b].at[pl.ds(G1, G2)]],
                         pbuf[b].at[pl.ds(G1, G2)], gsem[b])

    def wait_gather(b):
        pltpu.make_async_copy(table2_h.at[idxp[b].at[pl.ds(0, G1)]],
                              pbuf[b].at[pl.ds(0, G1)], gsem[b]).wait()
        pltpu.make_async_copy(table2_h.at[idxp[b].at[pl.ds(G1, G2)]],
                              pbuf[b].at[pl.ds(G1, G2)], gsem[b]).wait()

    def issue_wb(c, b):
        pltpu.async_copy(obuf[b], out_h.at[bbase + c], wsem[b])

    def wait_wb(b):
        pltpu.make_async_copy(obuf[b], out_h.at[bbase], wsem[b]).wait()

    def select_add(b):
        pb = pbuf[b]
        ob = obuf[b]
        xb = xc[b]

        @pl.loop(0, SEQ, unroll=8)
        def _row(r):
            xv = xb[pl.ds(r, LANES)]
            off = (xv[0] & 1) * D
            for j in range(D // LANES):
                dst = pl.ds(j * LANES, LANES)
                ob[r, dst] = pb[r, pl.ds(off + j * LANES, LANES)] + \
                    pe_v[r, dst]

    def slot(c, b, prefetch, wait_prev_wb, restage_x=True):
        b2 = 1 - b
        if prefetch:
            wait_xc(b2)
            stage_idx(b2)
            issue_gather(b2)
        wait_gather(b)
        if wait_prev_wb:
            wait_wb(b)
        select_add(b)
        if prefetch and restage_x:
            issue_xc(c + 2, b)
        issue_wb(c, b)

    issue_xc(0, 0)
    wait_xc(0)
    stage_idx(0)
    issue_gather(0)
    issue_xc(1, 1)

    slot(0, 0, True, False)
    slot(1, 1, True, False)

    @pl.loop(1, CHUNKS // 2 - 1)
    def _group(g):
        slot(2 * g, 0, True, True)
        slot(2 * g + 1, 1, True, True)

    slot(CHUNKS - 2, 0, True, True, restage_x=False)
    slot(CHUNKS - 1, 1, False, True)

    wait_wb(0)
    wait_wb(1)


@jax.jit
def _run(x_flat, table2, pe_seq):
    mesh = plsc.VectorSubcoreMesh(
        core_axis_name="c", subcore_axis_name="s", num_cores=NC,
        num_subcores=NS)
    grid_kernel = pl.kernel(
        _sc_body,
        out_type=jax.ShapeDtypeStruct((BATCH, SEQ, D), jnp.float32),
        mesh=mesh,
        scratch_types=(
            [pltpu.VMEM((SEQ, D), jnp.float32)]
            + [pltpu.VMEM((XC,), jnp.int32) for _ in range(2)]
            + [pltpu.VMEM((NV * LANES,), jnp.int32) for _ in range(2)]
            + [pltpu.VMEM((SEQ, 2 * D), jnp.float32) for _ in range(2)]
            + [pltpu.VMEM((SEQ, D), jnp.float32) for _ in range(2)]
            + [pltpu.SemaphoreType.DMA for _ in range(6)]
        ),
    )
    return grid_kernel(x_flat, table2, pe_seq)


def kernel(x, table, pe):
    x_flat = x.reshape(ROWS)
    table2 = table.reshape(table.shape[0] // 2, 2 * D)
    return _run(x_flat, table2, pe[:SEQ])


# speedup vs baseline: 1.0957x; 1.0957x over previous
import jax
import jax.numpy as jnp
from jax import lax
from jax.experimental import pallas as pl
from jax.experimental.pallas import tpu as pltpu
from jax.experimental.pallas import tpu_sc as plsc

BATCH = 4096
SEQ = 200
D = 64
NC = 2
NS = 16
NW = NC * NS
ROWS = BATCH * SEQ
PAIRS = ROWS // 2
RPW = ROWS // NW
PPW = PAIRS // NW
CHUNKS = RPW // SEQ
QC = SEQ // 2
G1 = 104
G2 = SEQ - G1
LANES = 16


def _sc_body(idx_h, table_h, pe_h, out_h, idx_v, pe_v,
             pb0, pb1, ob0, ob1, g0, g1s, w0, w1):
    pbuf = (pb0, pb1)
    obuf = (ob0, ob1)
    gsem = (g0, g1s)
    wsem = (w0, w1)

    wid = lax.axis_index("s") * NC + lax.axis_index("c")
    rbase = wid * RPW
    qbase = wid * PPW

    pltpu.sync_copy(idx_h.at[pl.ds(rbase, RPW)], idx_v)
    pltpu.sync_copy(pe_h, pe_v)

    def issue_gather(c, b):
        row0 = c * SEQ
        pltpu.async_copy(table_h.at[idx_v.at[pl.ds(row0, G1)]],
                         pbuf[b].at[pl.ds(0, G1)], gsem[b])
        pltpu.async_copy(table_h.at[idx_v.at[pl.ds(row0 + G1, G2)]],
                         pbuf[b].at[pl.ds(G1, G2)], gsem[b])

    def wait_gather(b):
        pltpu.make_async_copy(table_h.at[idx_v.at[pl.ds(0, G1)]],
                              pbuf[b].at[pl.ds(0, G1)], gsem[b]).wait()
        pltpu.make_async_copy(table_h.at[idx_v.at[pl.ds(0, G2)]],
                              pbuf[b].at[pl.ds(G1, G2)], gsem[b]).wait()

    def issue_wb(c, b):
        pltpu.async_copy(obuf[b], out_h.at[pl.ds(qbase + c * QC, QC)],
                         wsem[b])

    def wait_wb(b):
        pltpu.make_async_copy(obuf[b], out_h.at[pl.ds(qbase, QC)],
                              wsem[b]).wait()

    def add_pack(b):
        pb = pbuf[b]
        ob = obuf[b]

        @pl.loop(0, QC, unroll=4)
        def _pair(q):
            r = 2 * q
            for j in range(D // LANES):
                sl = pl.ds(j * LANES, LANES)
                sr = pl.ds(D + j * LANES, LANES)
                ob[q, sl] = pb[r, sl] + pe_v[q, sl]
                ob[q, sr] = pb[r + 1, sl] + pe_v[q, sr]

    def slot(c, b, prefetch, wait_prev_wb):
        if prefetch:
            issue_gather(c + 1, 1 - b)
        wait_gather(b)
        if wait_prev_wb:
            wait_wb(b)
        add_pack(b)
        issue_wb(c, b)

    issue_gather(0, 0)

    slot(0, 0, True, False)
    slot(1, 1, True, False)

    @pl.loop(1, CHUNKS // 2 - 1)
    def _group(g):
        slot(2 * g, 0, True, True)
        slot(2 * g + 1, 1, True, True)

    slot(CHUNKS - 2, 0, True, True)
    slot(CHUNKS - 1, 1, False, True)

    wait_wb(0)
    wait_wb(1)


@jax.jit
def _run(x_flat, table, pe_pair):
    mesh = plsc.VectorSubcoreMesh(
        core_axis_name="c", subcore_axis_name="s", num_cores=NC,
        num_subcores=NS)
    grid_kernel = pl.kernel(
        _sc_body,
        out_type=jax.ShapeDtypeStruct((PAIRS, 2 * D), jnp.float32),
        mesh=mesh,
        scratch_types=(
            [pltpu.VMEM((RPW,), jnp.int32),
             pltpu.VMEM((QC, 2 * D), jnp.float32)]
            + [pltpu.VMEM((SEQ, D), jnp.float32) for _ in range(2)]
            + [pltpu.VMEM((QC, 2 * D), jnp.float32) for _ in range(2)]
            + [pltpu.SemaphoreType.DMA for _ in range(4)]
        ),
        compiler_params=pltpu.CompilerParams(use_tc_tiling_on_sc=False),
    )
    return grid_kernel(x_flat, table, pe_pair)


def kernel(x, table, pe):
    x_flat = x.reshape(ROWS)
    pe_pair = pe[:SEQ].reshape(QC, 2 * D)
    out2 = _run(x_flat, table, pe_pair)
    return out2.reshape(BATCH, SEQ, D)


# speedup vs baseline: 1.2341x; 1.1263x over previous
import jax
import jax.numpy as jnp
from jax import lax
from jax.experimental import pallas as pl
from jax.experimental.pallas import tpu as pltpu
from jax.experimental.pallas import tpu_sc as plsc

BATCH = 4096
SEQ = 200
D = 64
NC = 2
NS = 16
NW = NC * NS
ROWS = BATCH * SEQ
RPW = ROWS // NW
BPW = BATCH // NW
CHUNKS = RPW // SEQ
G1 = 104
G2 = SEQ - G1
LANES = 16


def _sc_body(idx_h, table_h, pe_h, out_h, idx_v, pe_v, buf, sem):
    wid = lax.axis_index("s") * NC + lax.axis_index("c")
    rbase = wid * RPW
    bbase = wid * BPW

    pltpu.sync_copy(idx_h.at[pl.ds(rbase, RPW)], idx_v)
    pltpu.sync_copy(pe_h, pe_v)

    @pl.loop(0, CHUNKS)
    def _chunk(c):
        row0 = c * SEQ
        h1 = pltpu.async_copy(
            table_h.at[idx_v.at[pl.ds(row0, G1)]],
            buf.at[pl.ds(0, G1)], sem)
        h2 = pltpu.async_copy(
            table_h.at[idx_v.at[pl.ds(row0 + G1, G2)]],
            buf.at[pl.ds(G1, G2)], sem)
        h1.wait()
        h2.wait()

        @pl.loop(0, SEQ)
        def _row(r):
            for j in range(D // LANES):
                sl = pl.ds(j * LANES, LANES)
                buf[r, sl] = buf[r, sl] + pe_v[r, sl]

        pltpu.sync_copy(buf, out_h.at[bbase + c])


@jax.jit
def _run(x_flat, table, pe_seq):
    mesh = plsc.VectorSubcoreMesh(
        core_axis_name="c", subcore_axis_name="s", num_cores=NC,
        num_subcores=NS)
    grid_kernel = pl.kernel(
        _sc_body,
        out_type=jax.ShapeDtypeStruct((BATCH, SEQ, D), jnp.float32),
        mesh=mesh,
        scratch_types=[
            pltpu.VMEM((RPW,), jnp.int32),
            pltpu.VMEM((SEQ, D), jnp.float32),
            pltpu.VMEM((SEQ, D), jnp.float32),
            pltpu.SemaphoreType.DMA,
        ],
        compiler_params=pltpu.CompilerParams(use_tc_tiling_on_sc=False),
    )
    return grid_kernel(x_flat, table, pe_seq)


def kernel(x, table, pe):
    x_flat = x.reshape(ROWS)
    return _run(x_flat, table, pe[:SEQ])


# speedup vs baseline: 1.2886x; 1.0442x over previous
import jax
import jax.numpy as jnp
from jax import lax
from jax.experimental import pallas as pl
from jax.experimental.pallas import tpu as pltpu
from jax.experimental.pallas import tpu_sc as plsc

BATCH = 4096
SEQ = 200
D = 64
NC = 2
NS = 16
NW = NC * NS
ROWS = BATCH * SEQ
RPW = ROWS // NW
BPW = BATCH // NW
CHUNKS = RPW // SEQ
G1 = 104
G2 = SEQ - G1
LANES = 16


def _sc_body(idx_h, table_h, pe_h, out_h, idx_v, pe_v, buf0, buf1,
             sem0, sem1):
    wid = lax.axis_index("s") * NC + lax.axis_index("c")
    rbase = wid * RPW
    bbase = wid * BPW

    pltpu.sync_copy(idx_h.at[pl.ds(rbase, RPW)], idx_v)
    pltpu.sync_copy(pe_h, pe_v)

    def issue(c, buf, sem):
        row0 = c * SEQ
        h1 = pltpu.async_copy(
            table_h.at[idx_v.at[pl.ds(row0, G1)]],
            buf.at[pl.ds(0, G1)], sem)
        h2 = pltpu.async_copy(
            table_h.at[idx_v.at[pl.ds(row0 + G1, G2)]],
            buf.at[pl.ds(G1, G2)], sem)
        return h1, h2

    def consume(c, buf, handles):
        handles[0].wait()
        handles[1].wait()

        @pl.loop(0, SEQ)
        def _row(r):
            for j in range(D // LANES):
                sl = pl.ds(j * LANES, LANES)
                buf[r, sl] = buf[r, sl] + pe_v[r, sl]

        pltpu.sync_copy(buf, out_h.at[bbase + c])

    @pl.loop(0, CHUNKS // 2)
    def _group(g):
        c0 = 2 * g
        ha = issue(c0, buf0, sem0)
        hb = issue(c0 + 1, buf1, sem1)
        consume(c0, buf0, ha)
        consume(c0 + 1, buf1, hb)


@jax.jit
def _run(x_flat, table, pe_seq):
    mesh = plsc.VectorSubcoreMesh(
        core_axis_name="c", subcore_axis_name="s", num_cores=NC,
        num_subcores=NS)
    grid_kernel = pl.kernel(
        _sc_body,
        out_type=jax.ShapeDtypeStruct((BATCH, SEQ, D), jnp.float32),
        mesh=mesh,
        scratch_types=[
            pltpu.VMEM((RPW,), jnp.int32),
            pltpu.VMEM((SEQ, D), jnp.float32),
            pltpu.VMEM((SEQ, D), jnp.float32),
            pltpu.VMEM((SEQ, D), jnp.float32),
            pltpu.SemaphoreType.DMA,
            pltpu.SemaphoreType.DMA,
        ],
        compiler_params=pltpu.CompilerParams(use_tc_tiling_on_sc=False),
    )
    return grid_kernel(x_flat, table, pe_seq)


def kernel(x, table, pe):
    x_flat = x.reshape(ROWS)
    return _run(x_flat, table, pe[:SEQ])
